# Initial kernel scaffold; baseline (speedup 1.0000x reference)
import jax
import jax.numpy as jnp
from jax.experimental import pallas as pl


def kernel(neighbors, images):
    raise NotImplementedError("write your pallas kernel here")


# speedup vs baseline: 4.3468x; 4.3468x over previous
import jax
import jax.numpy as jnp
from jax.experimental import pallas as pl
from jax.experimental.pallas import tpu as pltpu

_K = 200
_NSORT = 256
_VIEWS = 4
_ROWS_PER_STEP = 1024
_PAD_VAL = jnp.iinfo(jnp.int32).max


_G = 32
_L = 128


def _halfclean(t, k, j, asc):
    b_dim = t.shape[0]
    z = t.reshape(-1, 2 * j * _G, _L)
    a, b = z[:, : j * _G], z[:, j * _G :]
    lo, hi = jnp.minimum(a, b), jnp.maximum(a, b)
    pair = jnp.concatenate([lo, hi] if asc else [hi, lo], axis=1)
    return pair.reshape(b_dim, k * _G, _L)


def _bitonic_sort_grouped(x, n):
    k = 2
    while k <= n:
        j = k // 2
        while j >= 1:
            nb = n // k
            if nb == 1:
                x = _halfclean(x.reshape(1, n * _G, _L), k, j, True)
                x = x.reshape(n * _G, _L)
            else:
                y = x.reshape(nb // 2, 2 * k * _G, _L)
                asc = _halfclean(y[:, : k * _G], k, j, True)
                desc = _halfclean(y[:, k * _G :], k, j, False)
                x = jnp.concatenate([asc, desc], axis=1).reshape(n * _G, _L)
            j //= 2
        k *= 2
    return x


def _body(v0_ref, v1_ref, v2_ref, v3_ref, o_ref):
    x = jnp.concatenate(
        [r[:, 0] for r in (v0_ref, v1_ref, v2_ref, v3_ref)], axis=1
    )
    pad = jnp.full((_NSORT - _K,) + x.shape[1:], _PAD_VAL, jnp.int32)
    x = jnp.concatenate([x, pad], axis=0)
    s = _bitonic_sort_grouped(x.reshape(_NSORT * _G, _L), _NSORT)
    s = s.reshape(_NSORT, _G, _L)
    neq = (s[1:_K] != s[: _K - 1]).astype(jnp.float32)
    uniq = 1.0 + jnp.sum(neq, axis=0)
    per_view = uniq.reshape(_VIEWS, 8, 128)
    o_ref[0] = -0.25 * jnp.sum(per_view, axis=0)


def kernel(neighbors, images):
    del images
    n_rows, k = neighbors.shape
    batch = n_rows // _VIEWS
    n_steps = batch // _ROWS_PER_STEP
    nt = neighbors.reshape(_VIEWS * n_steps, _ROWS_PER_STEP, k)
    nt = nt.transpose(2, 0, 1).reshape(k, _VIEWS * n_steps, 8, 128)

    def vspec(v):
        return pl.BlockSpec(
            (k, 1, 8, 128), lambda b, v=v: (0, v * n_steps + b, 0, 0)
        )

    out = pl.pallas_call(
        _body,
        grid=(n_steps,),
        in_specs=[vspec(0), vspec(1), vspec(2), vspec(3)],
        out_specs=pl.BlockSpec((1, 8, 128), lambda b: (b, 0, 0)),
        out_shape=jax.ShapeDtypeStruct((n_steps, 8, 128), jnp.float32),
        compiler_params=pltpu.CompilerParams(
            dimension_semantics=("arbitrary",),
        ),
    )(nt, nt, nt, nt)
    return out.reshape(batch)


# speedup vs baseline: 4.3469x; 1.0000x over previous
import jax
import jax.numpy as jnp
from jax.experimental import pallas as pl
from jax.experimental.pallas import tpu as pltpu

_K = 200
_NSORT = 256
_VIEWS = 4
_ROWS_PER_STEP = 1024
_PAD_VAL = jnp.iinfo(jnp.int32).max


_G = 32
_L = 128


def _halfclean(t, k, j, asc):
    b_dim = t.shape[0]
    z = t.reshape(-1, 2 * j * _G, _L)
    a, b = z[:, : j * _G], z[:, j * _G :]
    lo, hi = jnp.minimum(a, b), jnp.maximum(a, b)
    pair = jnp.concatenate([lo, hi] if asc else [hi, lo], axis=1)
    return pair.reshape(b_dim, k * _G, _L)


def _bitonic_sort_grouped(x, n):
    k = 2
    while k <= n:
        j = k // 2
        while j >= 1:
            nb = n // k
            if nb == 1:
                x = _halfclean(x.reshape(1, n * _G, _L), k, j, True)
                x = x.reshape(n * _G, _L)
            else:
                y = x.reshape(nb // 2, 2 * k * _G, _L)
                asc = _halfclean(y[:, : k * _G], k, j, True)
                desc = _halfclean(y[:, k * _G :], k, j, False)
                x = jnp.concatenate([asc, desc], axis=1).reshape(n * _G, _L)
            j //= 2
        k *= 2
    return x


def _body(v0_ref, v1_ref, v2_ref, v3_ref, o_ref):
    x = jnp.concatenate(
        [r[:, 0] for r in (v0_ref, v1_ref, v2_ref, v3_ref)], axis=1
    )
    pad = jnp.full((_NSORT - _K,) + x.shape[1:], _PAD_VAL, jnp.int32)
    x = jnp.concatenate([x, pad], axis=0)
    s = _bitonic_sort_grouped(x.reshape(_NSORT * _G, _L), _NSORT)
    s = s.reshape(_NSORT, _G, _L)
    neq = (s[1:_K] != s[: _K - 1]).astype(jnp.float32)
    uniq = 1.0 + jnp.sum(neq, axis=0)
    per_view = uniq.reshape(_VIEWS, 8, 128)
    o_ref[0] = -0.25 * jnp.sum(per_view, axis=0)


def kernel(neighbors, images):
    del images
    n_rows, k = neighbors.shape
    batch = n_rows // _VIEWS
    n_steps = batch // _ROWS_PER_STEP
    nt = neighbors.reshape(_VIEWS * n_steps, _ROWS_PER_STEP, k)
    nt = nt.transpose(2, 0, 1).reshape(k, _VIEWS * n_steps, 8, 128)

    def vspec(v):
        return pl.BlockSpec(
            (k, 1, 8, 128), lambda b, v=v: (0, v * n_steps + b, 0, 0)
        )

    out = pl.pallas_call(
        _body,
        grid=(n_steps,),
        in_specs=[vspec(0), vspec(1), vspec(2), vspec(3)],
        out_specs=pl.BlockSpec((1, 8, 128), lambda b: (b, 0, 0)),
        out_shape=jax.ShapeDtypeStruct((n_steps, 8, 128), jnp.float32),
        compiler_params=pltpu.CompilerParams(
            dimension_semantics=("parallel",),
        ),
    )(nt, nt, nt, nt)
    return out.reshape(batch)


# speedup vs baseline: 5.2334x; 1.2039x over previous
import jax
import jax.numpy as jnp
from jax.experimental import pallas as pl
from jax.experimental.pallas import tpu as pltpu

_K = 200
_NSORT = 256
_VIEWS = 4
_ROWS_PER_STEP = 1024
_PAD_VAL = jnp.iinfo(jnp.int32).max


_G = 32
_L = 128


def _halfclean(t, k, j, asc):
    b_dim = t.shape[0]
    z = t.reshape(-1, 2 * j * _G, _L)
    a, b = z[:, : j * _G], z[:, j * _G :]
    lo, hi = jnp.minimum(a, b), jnp.maximum(a, b)
    pair = jnp.concatenate([lo, hi] if asc else [hi, lo], axis=1)
    return pair.reshape(b_dim, k * _G, _L)


def _bitonic_sort_grouped(x, n):
    k = 2
    while k <= n:
        j = k // 2
        while j >= 1:
            nb = n // k
            if nb == 1:
                x = _halfclean(x.reshape(1, n * _G, _L), k, j, True)
                x = x.reshape(n * _G, _L)
            else:
                y = x.reshape(nb // 2, 2 * k * _G, _L)
                asc = _halfclean(y[:, : k * _G], k, j, True)
                desc = _halfclean(y[:, k * _G :], k, j, False)
                x = jnp.concatenate([asc, desc], axis=1).reshape(n * _G, _L)
            j //= 2
        k *= 2
    return x


def _body(v0_ref, v1_ref, v2_ref, v3_ref, o_ref):
    views = []
    for r in (v0_ref, v1_ref, v2_ref, v3_ref):
        nat = r[...].reshape(8, _L, _K)
        views.append(jnp.transpose(nat, (2, 0, 1)))
    x = jnp.concatenate(views, axis=1)
    pad = jnp.full((_NSORT - _K,) + x.shape[1:], _PAD_VAL, jnp.int32)
    x = jnp.concatenate([x, pad], axis=0)
    s = _bitonic_sort_grouped(x.reshape(_NSORT * _G, _L), _NSORT)
    s = s.reshape(_NSORT, _G, _L)
    neq = (s[1:_K] != s[: _K - 1]).astype(jnp.float32)
    uniq = 1.0 + jnp.sum(neq, axis=0)
    per_view = uniq.reshape(_VIEWS, 8, 128)
    o_ref[0] = -0.25 * jnp.sum(per_view, axis=0)


def kernel(neighbors, images):
    del images
    n_rows, k = neighbors.shape
    batch = n_rows // _VIEWS
    n_steps = batch // _ROWS_PER_STEP

    def vspec(v):
        return pl.BlockSpec(
            (_ROWS_PER_STEP, k), lambda b, v=v: (v * n_steps + b, 0)
        )

    out = pl.pallas_call(
        _body,
        grid=(n_steps,),
        in_specs=[vspec(0), vspec(1), vspec(2), vspec(3)],
        out_specs=pl.BlockSpec((1, 8, 128), lambda b: (b, 0, 0)),
        out_shape=jax.ShapeDtypeStruct((n_steps, 8, 128), jnp.float32),
        compiler_params=pltpu.CompilerParams(
            dimension_semantics=("parallel",),
        ),
    )(neighbors, neighbors, neighbors, neighbors)
    return out.reshape(batch)


# speedup vs baseline: 5.5887x; 1.0679x over previous
import jax
import jax.numpy as jnp
from jax.experimental import pallas as pl
from jax.experimental.pallas import tpu as pltpu

_K = 200
_NSORT = 256
_VIEWS = 4
_ROWS_PER_STEP = 1024
_PAD_F = 2.0**18


_G = 32
_L = 128


def _halfclean(t, k, j, asc):
    b_dim = t.shape[0]
    z = t.reshape(-1, 2 * j * _G, _L)
    a, b = z[:, : j * _G], z[:, j * _G :]
    lo, hi = jnp.minimum(a, b), jnp.maximum(a, b)
    pair = jnp.concatenate([lo, hi] if asc else [hi, lo], axis=1)
    return pair.reshape(b_dim, k * _G, _L)


def _bitonic_sort_grouped(x, n):
    k = 2
    while k <= n:
        j = k // 2
        while j >= 1:
            nb = n // k
            if nb == 1:
                x = _halfclean(x.reshape(1, n * _G, _L), k, j, True)
                x = x.reshape(n * _G, _L)
            else:
                y = x.reshape(nb // 2, 2 * k * _G, _L)
                asc = _halfclean(y[:, : k * _G], k, j, True)
                desc = _halfclean(y[:, k * _G :], k, j, False)
                x = jnp.concatenate([asc, desc], axis=1).reshape(n * _G, _L)
            j //= 2
        k *= 2
    return x


def _body(v0_ref, v1_ref, v2_ref, v3_ref, o_ref):
    views = []
    for r in (v0_ref, v1_ref, v2_ref, v3_ref):
        nat = r[...].reshape(8, _L, _K)
        views.append(jnp.transpose(nat, (2, 0, 1)))
    x = jnp.concatenate(views, axis=1)
    x = x.astype(jnp.float32)
    pad = jnp.full((_NSORT - _K,) + x.shape[1:], _PAD_F, jnp.float32)
    x = jnp.concatenate([x, pad], axis=0)
    s = _bitonic_sort_grouped(x.reshape(_NSORT * _G, _L), _NSORT)
    s = s.reshape(_NSORT, _G, _L)
    neq = (s[1:_K] != s[: _K - 1]).astype(jnp.float32)
    uniq = 1.0 + jnp.sum(neq, axis=0)
    per_view = uniq.reshape(_VIEWS, 8, 128)
    o_ref[0] = -0.25 * jnp.sum(per_view, axis=0)


def kernel(neighbors, images):
    del images
    n_rows, k = neighbors.shape
    batch = n_rows // _VIEWS
    n_steps = batch // _ROWS_PER_STEP

    def vspec(v):
        return pl.BlockSpec(
            (_ROWS_PER_STEP, k), lambda b, v=v: (v * n_steps + b, 0)
        )

    out = pl.pallas_call(
        _body,
        grid=(n_steps,),
        in_specs=[vspec(0), vspec(1), vspec(2), vspec(3)],
        out_specs=pl.BlockSpec((1, 8, 128), lambda b: (b, 0, 0)),
        out_shape=jax.ShapeDtypeStruct((n_steps, 8, 128), jnp.float32),
        compiler_params=pltpu.CompilerParams(
            dimension_semantics=("parallel",),
        ),
    )(neighbors, neighbors, neighbors, neighbors)
    return out.reshape(batch)


# speedup vs baseline: 5.6032x; 1.0026x over previous
import jax
import jax.numpy as jnp
from jax.experimental import pallas as pl
from jax.experimental.pallas import tpu as pltpu

_K = 200
_NSORT = 256
_VIEWS = 4
_ROWS_PER_STEP = 1024
_PAD_F = 2.0**18


_G = 32
_L = 128


def _halfclean(t, k, j, asc, g):
    b_dim = t.shape[0]
    z = t.reshape(-1, 2 * j * g, _L)
    a, b = z[:, : j * g], z[:, j * g :]
    lo, hi = jnp.minimum(a, b), jnp.maximum(a, b)
    pair = jnp.concatenate([lo, hi] if asc else [hi, lo], axis=1)
    return pair.reshape(b_dim, k * g, _L)


def _bitonic_sort_grouped(x, n, g):
    k = 2
    while k <= n:
        j = k // 2
        while j >= 1:
            nb = n // k
            if nb == 1:
                x = _halfclean(x.reshape(1, n * g, _L), k, j, True, g)
                x = x.reshape(n * g, _L)
            else:
                y = x.reshape(nb // 2, 2 * k * g, _L)
                asc = _halfclean(y[:, : k * g], k, j, True, g)
                desc = _halfclean(y[:, k * g :], k, j, False, g)
                x = jnp.concatenate([asc, desc], axis=1).reshape(n * g, _L)
            j //= 2
        k *= 2
    return x


def _body(v0_ref, v1_ref, v2_ref, v3_ref, o_ref):
    acc = None
    for r in (v0_ref, v1_ref, v2_ref, v3_ref):
        nat = r[...].reshape(8, _L, _K)
        x = jnp.transpose(nat, (2, 0, 1))
        x = x.astype(jnp.float32)
        pad = jnp.full((_NSORT - _K, 8, _L), _PAD_F, jnp.float32)
        x = jnp.concatenate([x, pad], axis=0)
        s = _bitonic_sort_grouped(x.reshape(_NSORT * 8, _L), _NSORT, 8)
        s = s.reshape(_NSORT, 8, _L)
        neq = (s[1:_K] != s[: _K - 1]).astype(jnp.float32)
        uniq = 1.0 + jnp.sum(neq, axis=0)
        acc = uniq if acc is None else acc + uniq
    o_ref[0] = -0.25 * acc


def kernel(neighbors, images):
    del images
    n_rows, k = neighbors.shape
    batch = n_rows // _VIEWS
    n_steps = batch // _ROWS_PER_STEP

    def vspec(v):
        return pl.BlockSpec(
            (_ROWS_PER_STEP, k), lambda b, v=v: (v * n_steps + b, 0)
        )

    out = pl.pallas_call(
        _body,
        grid=(n_steps,),
        in_specs=[vspec(0), vspec(1), vspec(2), vspec(3)],
        out_specs=pl.BlockSpec((1, 8, 128), lambda b: (b, 0, 0)),
        out_shape=jax.ShapeDtypeStruct((n_steps, 8, 128), jnp.float32),
        compiler_params=pltpu.CompilerParams(
            dimension_semantics=("parallel",),
        ),
    )(neighbors, neighbors, neighbors, neighbors)
    return out.reshape(batch)


# speedup vs baseline: 5.6245x; 1.0038x over previous
import jax
import jax.numpy as jnp
from jax.experimental import pallas as pl
from jax.experimental.pallas import tpu as pltpu

_K = 200
_NSORT = 256
_VIEWS = 4
_ROWS_PER_STEP = 1024
_PAD_F = 2.0**18


_G = 32
_L = 128


def _halfclean(t, k, j, asc, g):
    b_dim = t.shape[0]
    z = t.reshape(-1, 2 * j * g, _L)
    a, b = z[:, : j * g], z[:, j * g :]
    lo, hi = jnp.minimum(a, b), jnp.maximum(a, b)
    pair = jnp.concatenate([lo, hi] if asc else [hi, lo], axis=1)
    return pair.reshape(b_dim, k * g, _L)


def _bitonic_sort_grouped(x, n, g):
    k = 2
    while k <= n:
        j = k // 2
        while j >= 1:
            nb = n // k
            if nb == 1:
                x = _halfclean(x.reshape(1, n * g, _L), k, j, True, g)
                x = x.reshape(n * g, _L)
            else:
                y = x.reshape(nb // 2, 2 * k * g, _L)
                asc = _halfclean(y[:, : k * g], k, j, True, g)
                desc = _halfclean(y[:, k * g :], k, j, False, g)
                x = jnp.concatenate([asc, desc], axis=1).reshape(n * g, _L)
            j //= 2
        k *= 2
    return x


def _body(x_ref, o_ref):
    nat = x_ref[...].reshape(8, _L, _K)
    x = jnp.transpose(nat, (2, 0, 1))
    x = x.astype(jnp.float32)
    pad = jnp.full((_NSORT - _K, 8, _L), _PAD_F, jnp.float32)
    x = jnp.concatenate([x, pad], axis=0)
    s = _bitonic_sort_grouped(x.reshape(_NSORT * 8, _L), _NSORT, 8)
    s = s.reshape(_NSORT, 8, _L)
    neq = (s[1:_K] != s[: _K - 1]).astype(jnp.float32)
    uniq = 1.0 + jnp.sum(neq, axis=0)
    contrib = -0.25 * uniq

    @pl.when(pl.program_id(1) == 0)
    def _init():
        o_ref[0] = contrib

    @pl.when(pl.program_id(1) > 0)
    def _accum():
        o_ref[0] += contrib


def kernel(neighbors, images):
    del images
    n_rows, k = neighbors.shape
    batch = n_rows // _VIEWS
    n_steps = batch // _ROWS_PER_STEP

    out = pl.pallas_call(
        _body,
        grid=(n_steps, _VIEWS),
        in_specs=[
            pl.BlockSpec(
                (_ROWS_PER_STEP, k), lambda b, v: (v * n_steps + b, 0)
            )
        ],
        out_specs=pl.BlockSpec((1, 8, 128), lambda b, v: (b, 0, 0)),
        out_shape=jax.ShapeDtypeStruct((n_steps, 8, 128), jnp.float32),
        compiler_params=pltpu.CompilerParams(
            dimension_semantics=("parallel", "arbitrary"),
        ),
    )(neighbors)
    return out.reshape(batch)
